# baseline (speedup 1.0000x reference)
import functools

import jax
import jax.numpy as jnp
from jax import lax
from jax.experimental import pallas as pl
from jax.experimental.pallas import tpu as pltpu
from jax.experimental.pallas import tpu_sc as plsc

N = 10000
D = 128
P = 16
E = 320000
F = 640000
NC, NS, L = 2, 16, 16
NW = NC * NS
EW = E // NW
FW = F // NW
C = 80
RT = N // NS

_mesh = plsc.VectorSubcoreMesh(
    core_axis_name="c", subcore_axis_name="s", num_cores=NC, num_subcores=NS)

_f32 = jnp.float32
_i32 = jnp.int32


def _iota16():
    return lax.iota(_i32, L)


RT8 = 632
RTL = N - (NS - 1) * RT8


def _for_tile_span(s, fn):
    @pl.when(s < NS - 1)
    def _():
        fn(pl.multiple_of(s * RT8, 8), RT8)

    @pl.when(s == NS - 1)
    def _():
        fn((NS - 1) * RT8, RTL)


def _bounce_in(src_hbm, dst_sh, bounce_v, row0, nrows):
    for o in range(0, (nrows // C) * C, C):
        pltpu.sync_copy(src_hbm.at[pl.ds(row0 + o, C)], bounce_v)
        pltpu.sync_copy(bounce_v, dst_sh.at[pl.ds(row0 + o, C)])
    rem = nrows % C
    if rem:
        o = (nrows // C) * C
        pltpu.sync_copy(src_hbm.at[pl.ds(row0 + o, rem)], bounce_v.at[pl.ds(0, rem)])
        pltpu.sync_copy(bounce_v.at[pl.ds(0, rem)], dst_sh.at[pl.ds(row0 + o, rem)])


def _bounce_out(src_sh, dst_hbm, bounce_v, row0, nrows):
    for o in range(0, (nrows // C) * C, C):
        pltpu.sync_copy(src_sh.at[pl.ds(row0 + o, C)], bounce_v)
        pltpu.sync_copy(bounce_v, dst_hbm.at[pl.ds(row0 + o, C)])
    rem = nrows % C
    if rem:
        o = (nrows // C) * C
        pltpu.sync_copy(src_sh.at[pl.ds(row0 + o, rem)], bounce_v.at[pl.ds(0, rem)])
        pltpu.sync_copy(bounce_v.at[pl.ds(0, rem)], dst_hbm.at[pl.ds(row0 + o, rem)])


def _zero_rows(shared, src_v, row0, nrows):
    for k in range(nrows // C):
        pltpu.sync_copy(src_v, shared.at[pl.ds(row0 + k * C, C)])
    rem = nrows % C
    if rem:
        pltpu.sync_copy(src_v.at[pl.ds(0, rem)],
                        shared.at[pl.ds(row0 + (nrows // C) * C, rem)])


def _edge_stats_body(pos_hbm, row_hbm, col_hbm, src_hbm, dst_hbm,
                     rel2_hbm, deg_hbm, pdist_hbm,
                     pos_sh, posr_v, posc_v, hist_v, idxr_v, idxc_v,
                     rel2_v, sem):
    c = lax.axis_index("c")
    s = lax.axis_index("s")
    wid = s * NC + c

    def zhist(i, _):
        hist_v[pl.ds(i * L, L)] = jnp.zeros((L,), _f32)
        return 0
    lax.fori_loop(0, N // L, zhist, 0)

    _for_tile_span(s, lambda row0, nrows: _bounce_in(
        pos_hbm, pos_sh, posr_v, row0, nrows))
    plsc.subcore_barrier()

    iota = _iota16()
    ones = jnp.ones((L,), _f32)

    def run(a_hbm, b_hbm, out_hbm, base, nch, with_hist):
        def chunk(j, _):
            b = base + j * C
            pltpu.sync_copy(a_hbm.at[pl.ds(b, C)], idxr_v)
            pltpu.sync_copy(b_hbm.at[pl.ds(b, C)], idxc_v)
            d1 = pltpu.async_copy(pos_sh.at[idxr_v], posr_v, sem)
            d2 = pltpu.async_copy(pos_sh.at[idxc_v], posc_v, sem)
            d1.wait()
            d2.wait()

            def group(g, _):
                evec = g * L + iota
                acc = jnp.zeros((L,), _f32)
                for k in range(P):
                    kvec = jnp.full((L,), k, _i32)
                    dv = (plsc.load_gather(posr_v, [evec, kvec])
                          - plsc.load_gather(posc_v, [evec, kvec]))
                    acc = acc + dv * dv
                rel2_v[pl.ds(g * L, L)] = acc
                if with_hist:
                    plsc.addupdate_scatter(hist_v, [idxc_v[pl.ds(g * L, L)]], ones)
                return 0
            lax.fori_loop(0, C // L, group, 0)
            pltpu.sync_copy(rel2_v, out_hbm.at[pl.ds(b, C)])
            return 0
        lax.fori_loop(0, nch, chunk, 0)

    run(row_hbm, col_hbm, rel2_hbm, wid * EW, EW // C, True)
    run(src_hbm, dst_hbm, pdist_hbm, wid * FW, FW // C, False)

    pltpu.sync_copy(hist_v, deg_hbm.at[pl.ds(wid * N, N)])


_edge_stats = pl.kernel(
    _edge_stats_body,
    out_type=(jax.ShapeDtypeStruct((E,), _f32),
              jax.ShapeDtypeStruct((NW * N,), _f32),
              jax.ShapeDtypeStruct((F,), _f32)),
    mesh=_mesh,
    compiler_params=pltpu.CompilerParams(needs_layout_passes=False),
    scratch_types=(
        pltpu.VMEM_SHARED((N, D), _f32),
        pltpu.VMEM((C, D), _f32),
        pltpu.VMEM((C, D), _f32),
        pltpu.VMEM((N,), _f32),
        pltpu.VMEM((C,), _i32),
        pltpu.VMEM((C,), _i32),
        pltpu.VMEM((C,), _f32),
        pltpu.SemaphoreType.DMA,
    ),
)


def _agg_body(xt_hbm, dinv_hbm, pe_hbm, row_hbm, col_hbm, out_hbm,
              shared, dinv_v,
              rows0, ir0, ic0, pe0, rows1, ir1, ic1, pe1, sem0, sem1):
    c = lax.axis_index("c")
    s = lax.axis_index("s")
    wid = s * NC + c

    def zrows(i, _):
        for k in range(D // L):
            rows0[i, pl.ds(k * L, L)] = jnp.zeros((L,), _f32)
        return 0
    lax.fori_loop(0, C, zrows, 0)

    _for_tile_span(s, lambda row0, nrows: _zero_rows(shared, rows0, row0, nrows))
    pltpu.sync_copy(dinv_hbm, dinv_v)
    plsc.subcore_barrier()

    ebase = wid * EW
    iota = _iota16()
    NCH = EW // C
    slots = ((rows0, ir0, ic0, pe0, sem0), (rows1, ir1, ic1, pe1, sem1))

    def issue(j, t):
        rows_v, idxr_v, idxc_v, pe_v, sem = slots[t]
        b = ebase + j * C
        pltpu.sync_copy(row_hbm.at[pl.ds(b, C)], idxr_v)
        pltpu.sync_copy(col_hbm.at[pl.ds(b, C)], idxc_v)
        pltpu.sync_copy(pe_hbm.at[pl.ds(b, C)], pe_v)
        pltpu.async_copy(xt_hbm.at[idxr_v], rows_v, sem)

    def work(j, t):
        rows_v, idxr_v, idxc_v, pe_v, sem = slots[t]
        pltpu.make_async_copy(xt_hbm.at[idxr_v], rows_v, sem).wait()

        def group(g, _):
            sl = pl.ds(g * L, L)
            rvec = idxr_v[sl]
            cvec = idxc_v[sl]
            coeff = (pe_v[sl]
                     * plsc.load_gather(dinv_v, [rvec])
                     * plsc.load_gather(dinv_v, [cvec]))
            for e16 in range(L):
                ce = coeff[e16]
                e = g * L + e16
                for k in range(D // L):
                    ksl = pl.ds(k * L, L)
                    rows_v[e, ksl] = rows_v[e, ksl] * ce
            return 0
        lax.fori_loop(0, C // L, group, 0)
        pltpu.sync_copy(rows_v, shared.at[idxc_v], add=True)

    issue(0, 0)

    def body(j2, _):
        j = j2 * 2
        issue(j + 1, 1)
        work(j, 0)

        @pl.when(j + 2 < NCH)
        def _():
            issue(j + 2, 0)
        work(j + 1, 1)
        return 0
    lax.fori_loop(0, NCH // 2, body, 0)
    if NCH % 2 == 1:
        work(NCH - 1, 0)

    plsc.subcore_barrier()
    _for_tile_span(s, lambda row0, nrows: _bounce_out(
        shared, out_hbm.at[c], rows0, row0, nrows))


_agg = pl.kernel(
    _agg_body,
    out_type=jax.ShapeDtypeStruct((NC, N, D), _f32),
    mesh=_mesh,
    compiler_params=pltpu.CompilerParams(needs_layout_passes=False),
    scratch_types=(
        pltpu.VMEM_SHARED((N, D), _f32),
        pltpu.VMEM((N,), _f32),
        pltpu.VMEM((C, D), _f32),
        pltpu.VMEM((C,), _i32),
        pltpu.VMEM((C,), _i32),
        pltpu.VMEM((C,), _f32),
        pltpu.VMEM((C, D), _f32),
        pltpu.VMEM((C,), _i32),
        pltpu.VMEM((C,), _i32),
        pltpu.VMEM((C,), _f32),
        pltpu.SemaphoreType.DMA,
        pltpu.SemaphoreType.DMA,
    ),
)


def _head_body(h_hbm, src_hbm, dst_hbm, pdist_hbm, fcp_hbm, out_hbm,
               hs0, hd0, is0, id0, pp0,
               hs1, hd1, is1, id1, pp1,
               out_v, fcp_v, semh0, semh1):
    c = lax.axis_index("c")
    s = lax.axis_index("s")
    wid = s * NC + c
    pltpu.sync_copy(fcp_hbm, fcp_v)
    fcp = fcp_v[...]
    w0 = fcp[0]
    w1 = fcp[1]
    bb = fcp[2]

    fbase = wid * FW
    iota = _iota16()
    NCH = FW // C
    slots = ((is0, id0, hs0, hd0, pp0, semh0),
             (is1, id1, hs1, hd1, pp1, semh1))

    def issue(j, t):
        iv, dv, hs, hd, pp, semh = slots[t]
        b = fbase + j * C
        pltpu.sync_copy(src_hbm.at[pl.ds(b, C)], iv)
        pltpu.sync_copy(dst_hbm.at[pl.ds(b, C)], dv)
        pltpu.sync_copy(pdist_hbm.at[pl.ds(b, C)], pp)
        pltpu.async_copy(h_hbm.at[iv], hs, semh)
        pltpu.async_copy(h_hbm.at[dv], hd, semh)

    def drain(t):
        iv, dv, hs, hd, pp, semh = slots[t]
        pltpu.make_async_copy(h_hbm.at[iv], hs, semh).wait()
        pltpu.make_async_copy(h_hbm.at[dv], hd, semh).wait()

    def compute(j, t):
        iv, dv, hs, hd, pp, semh = slots[t]
        b = fbase + j * C

        def group(g, _):
            evec = g * L + iota
            a0 = jnp.zeros((L,), _f32)
            a1 = jnp.zeros((L,), _f32)
            a2 = jnp.zeros((L,), _f32)
            a3 = jnp.zeros((L,), _f32)
            for k in range(0, D, 4):
                kv0 = jnp.full((L,), k, _i32)
                kv1 = jnp.full((L,), k + 1, _i32)
                kv2 = jnp.full((L,), k + 2, _i32)
                kv3 = jnp.full((L,), k + 3, _i32)
                a0 = a0 + plsc.load_gather(hs, [evec, kv0]) * plsc.load_gather(hd, [evec, kv0])
                a1 = a1 + plsc.load_gather(hs, [evec, kv1]) * plsc.load_gather(hd, [evec, kv1])
                a2 = a2 + plsc.load_gather(hs, [evec, kv2]) * plsc.load_gather(hd, [evec, kv2])
                a3 = a3 + plsc.load_gather(hs, [evec, kv3]) * plsc.load_gather(hd, [evec, kv3])
            acc = (a0 + a1) + (a2 + a3)
            sl = pl.ds(g * L, L)
            out_v[sl] = w0 * acc + w1 * pp[sl] + bb
            return 0
        lax.fori_loop(0, C // L, group, 0)
        pltpu.sync_copy(out_v, out_hbm.at[pl.ds(b, C)])

    issue(0, 0)

    def body(j2, _):
        j = j2 * 2
        issue(j + 1, 1)
        drain(0)
        compute(j, 0)

        @pl.when(j + 2 < NCH)
        def _():
            issue(j + 2, 0)
        drain(1)
        compute(j + 1, 1)
        return 0
    lax.fori_loop(0, NCH // 2, body, 0)


_head = pl.kernel(
    _head_body,
    out_type=jax.ShapeDtypeStruct((F,), _f32),
    mesh=_mesh,
    compiler_params=pltpu.CompilerParams(needs_layout_passes=False),
    scratch_types=(
        pltpu.VMEM((C, D), _f32),
        pltpu.VMEM((C, D), _f32),
        pltpu.VMEM((C,), _i32),
        pltpu.VMEM((C,), _i32),
        pltpu.VMEM((C,), _f32),
        pltpu.VMEM((C, D), _f32),
        pltpu.VMEM((C, D), _f32),
        pltpu.VMEM((C,), _i32),
        pltpu.VMEM((C,), _i32),
        pltpu.VMEM((C,), _f32),
        pltpu.VMEM((C,), _f32),
        pltpu.VMEM((L,), _f32),
        pltpu.SemaphoreType.DMA,
        pltpu.SemaphoreType.DMA,
    ),
)


def _mlp_pe(rel, m1w, m1b, m2w, m2b):
    acc = jnp.zeros_like(rel) + m2b[0, 0]
    for j in range(32):
        acc = acc + jnp.maximum(rel * m1w[0, j] + m1b[0, j], 0.0) * m2w[j, 0]
    return jax.nn.sigmoid(acc)


def _prep_body(degp, rel2, x, W1,
               m1w1, m1b1, m2w1, m2b1, m1w2, m1b2, m2w2, m2b2,
               dinv_o, xt1_o, pe1_o, pe2_o):
    deg = jnp.sum(degp[...], axis=0) + 1.0
    dinv_o[...] = jnp.broadcast_to(lax.rsqrt(deg)[:, None], (N, P))
    xt1_o[...] = jnp.dot(x[...], W1[...], preferred_element_type=_f32)
    rel = jnp.sqrt(rel2[...])
    pe1_o[...] = _mlp_pe(rel, m1w1[...], m1b1[...], m2w1[...], m2b1[...])
    pe2_o[...] = _mlp_pe(rel, m1w2[...], m1b2[...], m2w2[...], m2b2[...])


def _tc_prep(degp, rel2, x, W1, p1, p2):
    return pl.pallas_call(
        _prep_body,
        out_shape=(jax.ShapeDtypeStruct((N, P), _f32),
                   jax.ShapeDtypeStruct((N, D), _f32),
                   jax.ShapeDtypeStruct(rel2.shape, _f32),
                   jax.ShapeDtypeStruct(rel2.shape, _f32)),
    )(degp, rel2, x, W1, *p1, *p2)


def _self_pe(m1b, m2w, m2b):
    return jax.nn.sigmoid(jnp.sum(jnp.maximum(m1b[...][0, :], 0.0) * m2w[...][:, 0])
                          + m2b[0, 0])


def _finish_mm_body(aggp0, aggp1, xt, dinv, b, m1b, m2w, m2b, W, out_o):
    pe_self = _self_pe(m1b, m2w, m2b)
    d0 = dinv[...][:, 0:1]
    h = aggp0[...] + aggp1[...] + pe_self * d0 * d0 * xt[...] + b[...]
    out_o[...] = jnp.dot(h, W[...], preferred_element_type=_f32)


def _finish_body(aggp0, aggp1, xt, dinv, b, m1b, m2w, m2b, out_o):
    pe_self = _self_pe(m1b, m2w, m2b)
    d0 = dinv[...][:, 0:1]
    out_o[...] = aggp0[...] + aggp1[...] + pe_self * d0 * d0 * xt[...] + b[...]


def _tc_finish(aggp0, aggp1, xt, dinv, b, m1b, m2w, m2b, W=None):
    if W is None:
        return pl.pallas_call(
            _finish_body, out_shape=jax.ShapeDtypeStruct((N, D), _f32),
        )(aggp0, aggp1, xt, dinv, b, m1b, m2w, m2b)
    return pl.pallas_call(
        _finish_mm_body, out_shape=jax.ShapeDtypeStruct((N, D), _f32),
    )(aggp0, aggp1, xt, dinv, b, m1b, m2w, m2b, W)


def kernel(x, pos, pos_edge_index, neg_edge_index, train_pos,
           W1, b1, m1w1, m1b1, m2w1, m2b1,
           W2, b2, m1w2, m1b2, m2w2, m2b2,
           fc_w, fc_b):
    row = train_pos[0].astype(_i32)
    col = train_pos[1].astype(_i32)
    pos128 = jnp.pad(pos, ((0, 0), (0, D - P)))

    ei = jnp.concatenate([pos_edge_index, neg_edge_index], axis=-1).astype(_i32)
    rel2, degp, pdist = _edge_stats(pos128, row, col, ei[0], ei[1])
    p1 = (m1w1, m1b1.reshape(1, 32), m2w1, m2b1.reshape(1, 1))
    p2 = (m1w2, m1b2.reshape(1, 32), m2w2, m2b2.reshape(1, 1))
    dinv16, xt1, pe1, pe2 = _tc_prep(
        degp.reshape(NW, N), rel2.reshape(E // D, D), x, W1, p1, p2)
    dinv = dinv16[:, 0]

    agg1 = _agg(xt1, dinv, pe1.reshape(-1), row, col)
    xt2 = _tc_finish(agg1[0], agg1[1], xt1, dinv16, b1.reshape(1, D), *p1[1:], W=W2)
    agg2 = _agg(xt2, dinv, pe2.reshape(-1), row, col)
    h2 = _tc_finish(agg2[0], agg2[1], xt2, dinv16, b2.reshape(1, D), *p2[1:])

    fcp = jnp.concatenate(
        [fc_w[:, 0], fc_b, jnp.zeros((13,), _f32)]).astype(_f32)
    out = _head(h2, ei[0], ei[1], pdist, fcp)
    return out.reshape(F, 1)


# speedup vs baseline: 1.0532x; 1.0532x over previous
import functools

import jax
import jax.numpy as jnp
from jax import lax
from jax.experimental import pallas as pl
from jax.experimental.pallas import tpu as pltpu
from jax.experimental.pallas import tpu_sc as plsc

N = 10000
D = 128
P = 16
E = 320000
F = 640000
NC, NS, L = 2, 16, 16
NW = NC * NS
EW = E // NW
FW = F // NW
C = 80
RT = N // NS

_mesh = plsc.VectorSubcoreMesh(
    core_axis_name="c", subcore_axis_name="s", num_cores=NC, num_subcores=NS)

_f32 = jnp.float32
_i32 = jnp.int32


def _iota16():
    return lax.iota(_i32, L)


RT8 = 632
RTL = N - (NS - 1) * RT8


def _for_tile_span(s, fn):
    @pl.when(s < NS - 1)
    def _():
        fn(pl.multiple_of(s * RT8, 8), RT8)

    @pl.when(s == NS - 1)
    def _():
        fn((NS - 1) * RT8, RTL)


def _bounce_in(src_hbm, dst_sh, bounce_v, row0, nrows):
    for o in range(0, (nrows // C) * C, C):
        pltpu.sync_copy(src_hbm.at[pl.ds(row0 + o, C)], bounce_v)
        pltpu.sync_copy(bounce_v, dst_sh.at[pl.ds(row0 + o, C)])
    rem = nrows % C
    if rem:
        o = (nrows // C) * C
        pltpu.sync_copy(src_hbm.at[pl.ds(row0 + o, rem)], bounce_v.at[pl.ds(0, rem)])
        pltpu.sync_copy(bounce_v.at[pl.ds(0, rem)], dst_sh.at[pl.ds(row0 + o, rem)])


def _bounce_out(src_sh, dst_hbm, bounce_v, row0, nrows):
    for o in range(0, (nrows // C) * C, C):
        pltpu.sync_copy(src_sh.at[pl.ds(row0 + o, C)], bounce_v)
        pltpu.sync_copy(bounce_v, dst_hbm.at[pl.ds(row0 + o, C)])
    rem = nrows % C
    if rem:
        o = (nrows // C) * C
        pltpu.sync_copy(src_sh.at[pl.ds(row0 + o, rem)], bounce_v.at[pl.ds(0, rem)])
        pltpu.sync_copy(bounce_v.at[pl.ds(0, rem)], dst_hbm.at[pl.ds(row0 + o, rem)])


def _zero_rows(shared, src_v, row0, nrows):
    for k in range(nrows // C):
        pltpu.sync_copy(src_v, shared.at[pl.ds(row0 + k * C, C)])
    rem = nrows % C
    if rem:
        pltpu.sync_copy(src_v.at[pl.ds(0, rem)],
                        shared.at[pl.ds(row0 + (nrows // C) * C, rem)])


def _edge_stats_body(pos_hbm, row_hbm, col_hbm, src_hbm, dst_hbm,
                     rel2_hbm, deg_hbm, pdist_hbm,
                     pos_sh, posr_v, posc_v, hist_v, idxr_v, idxc_v,
                     rel2_v, sem):
    c = lax.axis_index("c")
    s = lax.axis_index("s")
    wid = s * NC + c

    def zhist(i, _):
        hist_v[pl.ds(i * L, L)] = jnp.zeros((L,), _f32)
        return 0
    lax.fori_loop(0, N // L, zhist, 0)

    _for_tile_span(s, lambda row0, nrows: _bounce_in(
        pos_hbm, pos_sh, posr_v, row0, nrows))
    plsc.subcore_barrier()

    iota = _iota16()
    ones = jnp.ones((L,), _f32)

    def run(a_hbm, b_hbm, out_hbm, base, nch, with_hist):
        def chunk(j, _):
            b = base + j * C
            pltpu.sync_copy(a_hbm.at[pl.ds(b, C)], idxr_v)
            pltpu.sync_copy(b_hbm.at[pl.ds(b, C)], idxc_v)
            d1 = pltpu.async_copy(pos_sh.at[idxr_v], posr_v, sem)
            d2 = pltpu.async_copy(pos_sh.at[idxc_v], posc_v, sem)
            d1.wait()
            d2.wait()

            @plsc.parallel_loop(0, C // L, unroll=2)
            def group(g):
                evec = g * L + iota
                acc = jnp.zeros((L,), _f32)
                for k in range(P):
                    kvec = jnp.full((L,), k, _i32)
                    dv = (plsc.load_gather(posr_v, [evec, kvec])
                          - plsc.load_gather(posc_v, [evec, kvec]))
                    acc = acc + dv * dv
                rel2_v[pl.ds(g * L, L)] = acc
                if with_hist:
                    plsc.addupdate_scatter(hist_v, [idxc_v[pl.ds(g * L, L)]], ones)
            pltpu.sync_copy(rel2_v, out_hbm.at[pl.ds(b, C)])
            return 0
        lax.fori_loop(0, nch, chunk, 0)

    run(row_hbm, col_hbm, rel2_hbm, wid * EW, EW // C, True)
    run(src_hbm, dst_hbm, pdist_hbm, wid * FW, FW // C, False)

    pltpu.sync_copy(hist_v, deg_hbm.at[pl.ds(wid * N, N)])


_edge_stats = pl.kernel(
    _edge_stats_body,
    out_type=(jax.ShapeDtypeStruct((E,), _f32),
              jax.ShapeDtypeStruct((NW * N,), _f32),
              jax.ShapeDtypeStruct((F,), _f32)),
    mesh=_mesh,
    compiler_params=pltpu.CompilerParams(needs_layout_passes=False),
    scratch_types=(
        pltpu.VMEM_SHARED((N, D), _f32),
        pltpu.VMEM((C, D), _f32),
        pltpu.VMEM((C, D), _f32),
        pltpu.VMEM((N,), _f32),
        pltpu.VMEM((C,), _i32),
        pltpu.VMEM((C,), _i32),
        pltpu.VMEM((C,), _f32),
        pltpu.SemaphoreType.DMA,
    ),
)


def _agg_body(xt_hbm, dinv_hbm, pe_hbm, row_hbm, col_hbm, out_hbm,
              shared, dinv_v,
              rows0, ir0, ic0, pe0, rows1, ir1, ic1, pe1, sem0, sem1):
    c = lax.axis_index("c")
    s = lax.axis_index("s")
    wid = s * NC + c

    def zrows(i, _):
        for k in range(D // L):
            rows0[i, pl.ds(k * L, L)] = jnp.zeros((L,), _f32)
        return 0
    lax.fori_loop(0, C, zrows, 0)

    _for_tile_span(s, lambda row0, nrows: _zero_rows(shared, rows0, row0, nrows))
    pltpu.sync_copy(dinv_hbm, dinv_v)
    plsc.subcore_barrier()

    ebase = wid * EW
    iota = _iota16()
    NCH = EW // C
    slots = ((rows0, ir0, ic0, pe0, sem0), (rows1, ir1, ic1, pe1, sem1))

    def issue(j, t):
        rows_v, idxr_v, idxc_v, pe_v, sem = slots[t]
        b = ebase + j * C
        pltpu.sync_copy(row_hbm.at[pl.ds(b, C)], idxr_v)
        pltpu.sync_copy(col_hbm.at[pl.ds(b, C)], idxc_v)
        pltpu.sync_copy(pe_hbm.at[pl.ds(b, C)], pe_v)
        pltpu.async_copy(xt_hbm.at[idxr_v], rows_v, sem)

    def work(j, t):
        rows_v, idxr_v, idxc_v, pe_v, sem = slots[t]
        pltpu.make_async_copy(xt_hbm.at[idxr_v], rows_v, sem).wait()

        def group(g, _):
            sl = pl.ds(g * L, L)
            rvec = idxr_v[sl]
            cvec = idxc_v[sl]
            coeff = (pe_v[sl]
                     * plsc.load_gather(dinv_v, [rvec])
                     * plsc.load_gather(dinv_v, [cvec]))
            for e16 in range(L):
                ce = coeff[e16]
                e = g * L + e16
                for k in range(D // L):
                    ksl = pl.ds(k * L, L)
                    rows_v[e, ksl] = rows_v[e, ksl] * ce
            return 0
        lax.fori_loop(0, C // L, group, 0)
        pltpu.sync_copy(rows_v, shared.at[idxc_v], add=True)

    issue(0, 0)

    def body(j2, _):
        j = j2 * 2
        issue(j + 1, 1)
        work(j, 0)

        @pl.when(j + 2 < NCH)
        def _():
            issue(j + 2, 0)
        work(j + 1, 1)
        return 0
    lax.fori_loop(0, NCH // 2, body, 0)
    if NCH % 2 == 1:
        work(NCH - 1, 0)

    plsc.subcore_barrier()
    _for_tile_span(s, lambda row0, nrows: _bounce_out(
        shared, out_hbm.at[c], rows0, row0, nrows))


_agg = pl.kernel(
    _agg_body,
    out_type=jax.ShapeDtypeStruct((NC, N, D), _f32),
    mesh=_mesh,
    compiler_params=pltpu.CompilerParams(needs_layout_passes=False),
    scratch_types=(
        pltpu.VMEM_SHARED((N, D), _f32),
        pltpu.VMEM((N,), _f32),
        pltpu.VMEM((C, D), _f32),
        pltpu.VMEM((C,), _i32),
        pltpu.VMEM((C,), _i32),
        pltpu.VMEM((C,), _f32),
        pltpu.VMEM((C, D), _f32),
        pltpu.VMEM((C,), _i32),
        pltpu.VMEM((C,), _i32),
        pltpu.VMEM((C,), _f32),
        pltpu.SemaphoreType.DMA,
        pltpu.SemaphoreType.DMA,
    ),
)


def _head_body(h_hbm, src_hbm, dst_hbm, pdist_hbm, fcp_hbm, out_hbm,
               hs0, hd0, is0, id0, pp0,
               hs1, hd1, is1, id1, pp1,
               out_v, fcp_v, semh0, semh1):
    c = lax.axis_index("c")
    s = lax.axis_index("s")
    wid = s * NC + c
    pltpu.sync_copy(fcp_hbm, fcp_v)
    fcp = fcp_v[...]
    w0 = fcp[0]
    w1 = fcp[1]
    bb = fcp[2]

    fbase = wid * FW
    iota = _iota16()
    NCH = FW // C
    slots = ((is0, id0, hs0, hd0, pp0, semh0),
             (is1, id1, hs1, hd1, pp1, semh1))

    def issue(j, t):
        iv, dv, hs, hd, pp, semh = slots[t]
        b = fbase + j * C
        pltpu.sync_copy(src_hbm.at[pl.ds(b, C)], iv)
        pltpu.sync_copy(dst_hbm.at[pl.ds(b, C)], dv)
        pltpu.sync_copy(pdist_hbm.at[pl.ds(b, C)], pp)
        pltpu.async_copy(h_hbm.at[iv], hs, semh)
        pltpu.async_copy(h_hbm.at[dv], hd, semh)

    def drain(t):
        iv, dv, hs, hd, pp, semh = slots[t]
        pltpu.make_async_copy(h_hbm.at[iv], hs, semh).wait()
        pltpu.make_async_copy(h_hbm.at[dv], hd, semh).wait()

    def compute(j, t):
        iv, dv, hs, hd, pp, semh = slots[t]
        b = fbase + j * C

        @plsc.parallel_loop(0, C // L, unroll=2)
        def group(g):
            evec = g * L + iota
            a0 = jnp.zeros((L,), _f32)
            a1 = jnp.zeros((L,), _f32)
            a2 = jnp.zeros((L,), _f32)
            a3 = jnp.zeros((L,), _f32)
            for k in range(0, D, 4):
                kv0 = jnp.full((L,), k, _i32)
                kv1 = jnp.full((L,), k + 1, _i32)
                kv2 = jnp.full((L,), k + 2, _i32)
                kv3 = jnp.full((L,), k + 3, _i32)
                a0 = a0 + plsc.load_gather(hs, [evec, kv0]) * plsc.load_gather(hd, [evec, kv0])
                a1 = a1 + plsc.load_gather(hs, [evec, kv1]) * plsc.load_gather(hd, [evec, kv1])
                a2 = a2 + plsc.load_gather(hs, [evec, kv2]) * plsc.load_gather(hd, [evec, kv2])
                a3 = a3 + plsc.load_gather(hs, [evec, kv3]) * plsc.load_gather(hd, [evec, kv3])
            acc = (a0 + a1) + (a2 + a3)
            sl = pl.ds(g * L, L)
            out_v[sl] = w0 * acc + w1 * pp[sl] + bb
        pltpu.sync_copy(out_v, out_hbm.at[pl.ds(b, C)])

    issue(0, 0)

    def body(j2, _):
        j = j2 * 2
        issue(j + 1, 1)
        drain(0)
        compute(j, 0)

        @pl.when(j + 2 < NCH)
        def _():
            issue(j + 2, 0)
        drain(1)
        compute(j + 1, 1)
        return 0
    lax.fori_loop(0, NCH // 2, body, 0)


_head = pl.kernel(
    _head_body,
    out_type=jax.ShapeDtypeStruct((F,), _f32),
    mesh=_mesh,
    compiler_params=pltpu.CompilerParams(needs_layout_passes=False),
    scratch_types=(
        pltpu.VMEM((C, D), _f32),
        pltpu.VMEM((C, D), _f32),
        pltpu.VMEM((C,), _i32),
        pltpu.VMEM((C,), _i32),
        pltpu.VMEM((C,), _f32),
        pltpu.VMEM((C, D), _f32),
        pltpu.VMEM((C, D), _f32),
        pltpu.VMEM((C,), _i32),
        pltpu.VMEM((C,), _i32),
        pltpu.VMEM((C,), _f32),
        pltpu.VMEM((C,), _f32),
        pltpu.VMEM((L,), _f32),
        pltpu.SemaphoreType.DMA,
        pltpu.SemaphoreType.DMA,
    ),
)


def _mlp_pe(rel, m1w, m1b, m2w, m2b):
    acc = jnp.zeros_like(rel) + m2b[0, 0]
    for j in range(32):
        acc = acc + jnp.maximum(rel * m1w[0, j] + m1b[0, j], 0.0) * m2w[j, 0]
    return jax.nn.sigmoid(acc)


def _prep_body(degp, rel2, x, W1,
               m1w1, m1b1, m2w1, m2b1, m1w2, m1b2, m2w2, m2b2,
               dinv_o, xt1_o, pe1_o, pe2_o):
    deg = jnp.sum(degp[...], axis=0) + 1.0
    dinv_o[...] = jnp.broadcast_to(lax.rsqrt(deg)[:, None], (N, P))
    xt1_o[...] = jnp.dot(x[...], W1[...], preferred_element_type=_f32)
    rel = jnp.sqrt(rel2[...])
    pe1_o[...] = _mlp_pe(rel, m1w1[...], m1b1[...], m2w1[...], m2b1[...])
    pe2_o[...] = _mlp_pe(rel, m1w2[...], m1b2[...], m2w2[...], m2b2[...])


def _tc_prep(degp, rel2, x, W1, p1, p2):
    return pl.pallas_call(
        _prep_body,
        out_shape=(jax.ShapeDtypeStruct((N, P), _f32),
                   jax.ShapeDtypeStruct((N, D), _f32),
                   jax.ShapeDtypeStruct(rel2.shape, _f32),
                   jax.ShapeDtypeStruct(rel2.shape, _f32)),
    )(degp, rel2, x, W1, *p1, *p2)


def _self_pe(m1b, m2w, m2b):
    return jax.nn.sigmoid(jnp.sum(jnp.maximum(m1b[...][0, :], 0.0) * m2w[...][:, 0])
                          + m2b[0, 0])


def _finish_mm_body(aggp0, aggp1, xt, dinv, b, m1b, m2w, m2b, W, out_o):
    pe_self = _self_pe(m1b, m2w, m2b)
    d0 = dinv[...][:, 0:1]
    h = aggp0[...] + aggp1[...] + pe_self * d0 * d0 * xt[...] + b[...]
    out_o[...] = jnp.dot(h, W[...], preferred_element_type=_f32)


def _finish_body(aggp0, aggp1, xt, dinv, b, m1b, m2w, m2b, out_o):
    pe_self = _self_pe(m1b, m2w, m2b)
    d0 = dinv[...][:, 0:1]
    out_o[...] = aggp0[...] + aggp1[...] + pe_self * d0 * d0 * xt[...] + b[...]


def _tc_finish(aggp0, aggp1, xt, dinv, b, m1b, m2w, m2b, W=None):
    if W is None:
        return pl.pallas_call(
            _finish_body, out_shape=jax.ShapeDtypeStruct((N, D), _f32),
        )(aggp0, aggp1, xt, dinv, b, m1b, m2w, m2b)
    return pl.pallas_call(
        _finish_mm_body, out_shape=jax.ShapeDtypeStruct((N, D), _f32),
    )(aggp0, aggp1, xt, dinv, b, m1b, m2w, m2b, W)


def kernel(x, pos, pos_edge_index, neg_edge_index, train_pos,
           W1, b1, m1w1, m1b1, m2w1, m2b1,
           W2, b2, m1w2, m1b2, m2w2, m2b2,
           fc_w, fc_b):
    row = train_pos[0].astype(_i32)
    col = train_pos[1].astype(_i32)
    pos128 = jnp.pad(pos, ((0, 0), (0, D - P)))

    ei = jnp.concatenate([pos_edge_index, neg_edge_index], axis=-1).astype(_i32)
    rel2, degp, pdist = _edge_stats(pos128, row, col, ei[0], ei[1])
    p1 = (m1w1, m1b1.reshape(1, 32), m2w1, m2b1.reshape(1, 1))
    p2 = (m1w2, m1b2.reshape(1, 32), m2w2, m2b2.reshape(1, 1))
    dinv16, xt1, pe1, pe2 = _tc_prep(
        degp.reshape(NW, N), rel2.reshape(E // D, D), x, W1, p1, p2)
    dinv = dinv16[:, 0]

    agg1 = _agg(xt1, dinv, pe1.reshape(-1), row, col)
    xt2 = _tc_finish(agg1[0], agg1[1], xt1, dinv16, b1.reshape(1, D), *p1[1:], W=W2)
    agg2 = _agg(xt2, dinv, pe2.reshape(-1), row, col)
    h2 = _tc_finish(agg2[0], agg2[1], xt2, dinv16, b2.reshape(1, D), *p2[1:])

    fcp = jnp.concatenate(
        [fc_w[:, 0], fc_b, jnp.zeros((13,), _f32)]).astype(_f32)
    out = _head(h2, ei[0], ei[1], pdist, fcp)
    return out.reshape(F, 1)
